# Initial kernel scaffold; baseline (speedup 1.0000x reference)
import jax
import jax.numpy as jnp
from jax.experimental import pallas as pl


def kernel(features, edge_index, W, b):
    raise NotImplementedError("write your pallas kernel here")


# speedup vs baseline: 1.8889x; 1.8889x over previous
import functools

import jax
import jax.numpy as jnp
from jax import lax
from jax.experimental import pallas as pl
from jax.experimental.pallas import tpu as pltpu
from jax.experimental.pallas import tpu_sc as plsc

N = 10000
D = 128
E = 320000
NC = 2
NS = 16
NT = NC * NS
CPT = D // NT
N4 = N * CPT
EPT = E // NT
CH = 4000

_mesh = plsc.VectorSubcoreMesh(core_axis_name="c", subcore_axis_name="s")
_sc_params = pltpu.CompilerParams(needs_layout_passes=False)


@functools.partial(
    pl.kernel,
    out_type=jax.ShapeDtypeStruct((NT, N), jnp.float32),
    mesh=_mesh,
    compiler_params=_sc_params,
    scratch_types=[
        pltpu.VMEM((N,), jnp.float32),
        pltpu.VMEM((EPT,), jnp.int32),
    ],
)
def _deg_kernel(dst_hbm, out_hbm, cnt_v, dbuf_v):
    w = lax.axis_index("s") * NC + lax.axis_index("c")
    zeros = jnp.zeros((16,), jnp.float32)
    ones = jnp.full((16,), 1.0, jnp.float32)

    @pl.loop(0, N // 16)
    def _(i):
        cnt_v[pl.ds(i * 16, 16)] = zeros

    pltpu.sync_copy(dst_hbm.at[pl.ds(w * EPT, EPT)], dbuf_v)

    @pl.loop(0, EPT // 16)
    def _(g):
        d = dbuf_v[pl.ds(g * 16, 16)]
        plsc.addupdate_scatter(cnt_v, [d], ones)

    pltpu.sync_copy(cnt_v, out_hbm.at[w])


_BN = 1000


def _prep_body(x_ref, w_ref, b_ref, degt_ref, hs0_ref, f1_ref, f2_ref):
    deg = jnp.sum(degt_ref[...], axis=1, keepdims=True)
    n = lax.rsqrt(deg + 1.0)
    h0 = lax.dot_general(
        x_ref[...], w_ref[...], (((1,), (1,)), ((), ())),
        preferred_element_type=jnp.float32,
    ) + b_ref[...]
    hs0_ref[...] = h0 * n
    f1_ref[...] = n * n
    f2_ref[...] = n


def _tc_prep(features, W, b2, deg_t):
    return pl.pallas_call(
        _prep_body,
        grid=(N // _BN,),
        in_specs=[
            pl.BlockSpec((_BN, D), lambda i: (i, 0)),
            pl.BlockSpec((D, D), lambda i: (0, 0)),
            pl.BlockSpec((1, D), lambda i: (0, 0)),
            pl.BlockSpec((_BN, NT), lambda i: (i, 0)),
        ],
        out_specs=[
            pl.BlockSpec((_BN, D), lambda i: (i, 0)),
            pl.BlockSpec((_BN, 1), lambda i: (i, 0)),
            pl.BlockSpec((_BN, 1), lambda i: (i, 0)),
        ],
        out_shape=[
            jax.ShapeDtypeStruct((N, D), jnp.float32),
            jax.ShapeDtypeStruct((N, 1), jnp.float32),
            jax.ShapeDtypeStruct((N, 1), jnp.float32),
        ],
    )(features, W, b2, deg_t)


@functools.partial(
    pl.kernel,
    out_type=jax.ShapeDtypeStruct((NT, N4), jnp.float32),
    mesh=_mesh,
    compiler_params=_sc_params,
    scratch_types=[
        pltpu.VMEM((N4,), jnp.float32),
        pltpu.VMEM((N4,), jnp.float32),
        pltpu.VMEM((N,), jnp.float32),
        pltpu.VMEM((N,), jnp.float32),
        pltpu.VMEM((CH,), jnp.int32),
        pltpu.VMEM((CH,), jnp.int32),
    ],
)
def _prop_kernel(hs0_hbm, src_hbm, dst_hbm, f1_hbm, f2_hbm, out_hbm,
                 bufA, bufB, f1v, f2v, sbuf, dbuf):
    w = lax.axis_index("s") * NC + lax.axis_index("c")
    pltpu.sync_copy(hs0_hbm.at[w], bufA)
    pltpu.sync_copy(hs0_hbm.at[w], bufB)
    pltpu.sync_copy(f1_hbm, f1v)
    pltpu.sync_copy(f2_hbm, f2v)
    iota = lax.iota(jnp.int32, 16)

    def edge_pass(table, acc):
        @pl.loop(0, E // CH)
        def _(ci):
            pltpu.sync_copy(src_hbm.at[pl.ds(ci * CH, CH)], sbuf)
            pltpu.sync_copy(dst_hbm.at[pl.ds(ci * CH, CH)], dbuf)

            @pl.loop(0, CH // 16)
            def _(g):
                s = sbuf[pl.ds(g * 16, 16)]
                d = dbuf[pl.ds(g * 16, 16)]
                si = s * CPT
                di = d * CPT
                for j in range(CPT):
                    v = plsc.load_gather(table, [si + j])
                    plsc.addupdate_scatter(acc, [di + j], v)

    def scale(acc, fv):
        @pl.loop(0, N4 // 16)
        def _(g):
            base = g * 16
            ridx = lax.shift_right_logical(base + iota, 2)
            f = plsc.load_gather(fv, [ridx])
            acc[pl.ds(base, 16)] = acc[pl.ds(base, 16)] * f

    edge_pass(bufA, bufB)
    scale(bufB, f1v)
    edge_pass(bufB, bufA)
    scale(bufA, f2v)
    pltpu.sync_copy(bufA, out_hbm.at[w])


@jax.jit
def kernel(features, edge_index, W, b):
    src = edge_index[0]
    dst = edge_index[1]
    deg_part = _deg_kernel(dst)
    hs0, f1, f2 = _tc_prep(features, W, b.reshape(1, D), deg_part.T)
    hs0_l = hs0.reshape(N, NT, CPT).transpose(1, 0, 2).reshape(NT, N4)
    out_l = _prop_kernel(hs0_l, src, dst, f1.reshape(N), f2.reshape(N))
    return out_l.reshape(NT, N, CPT).transpose(1, 0, 2).reshape(N, D)


# speedup vs baseline: 2.2781x; 1.2061x over previous
import functools

import jax
import jax.numpy as jnp
from jax import lax
from jax.experimental import pallas as pl
from jax.experimental.pallas import tpu as pltpu
from jax.experimental.pallas import tpu_sc as plsc

N = 10000
D = 128
E = 320000
NC = 2
NS = 16
NT = NC * NS
CPT = D // NT
N4 = N * CPT
EPT = E // NT
CH = 4000
NCH = E // CH

_mesh = plsc.VectorSubcoreMesh(core_axis_name="c", subcore_axis_name="s")
_sc_params = pltpu.CompilerParams(needs_layout_passes=False)


@functools.partial(
    pl.kernel,
    out_type=jax.ShapeDtypeStruct((NT, N), jnp.float32),
    mesh=_mesh,
    compiler_params=_sc_params,
    scratch_types=[
        pltpu.VMEM((N,), jnp.float32),
        pltpu.VMEM((EPT,), jnp.int32),
    ],
)
def _deg_kernel(dst_hbm, out_hbm, cnt_v, dbuf_v):
    w = lax.axis_index("s") * NC + lax.axis_index("c")
    zeros = jnp.zeros((16,), jnp.float32)
    ones = jnp.full((16,), 1.0, jnp.float32)

    @functools.partial(plsc.parallel_loop, 0, N // 16, unroll=8)
    def _(i):
        cnt_v[pl.ds(i * 16, 16)] = zeros

    pltpu.sync_copy(dst_hbm.at[pl.ds(w * EPT, EPT)], dbuf_v)

    @pl.loop(0, EPT // 16, unroll=8)
    def _(g):
        d = dbuf_v[pl.ds(g * 16, 16)]
        plsc.addupdate_scatter(cnt_v, [d], ones)

    pltpu.sync_copy(cnt_v, out_hbm.at[w])


_BN = 1000


def _prep_body(x_ref, w_ref, b_ref, degt_ref, hs0_ref, f1_ref, f2_ref):
    deg = jnp.sum(degt_ref[...], axis=1, keepdims=True)
    n = lax.rsqrt(deg + 1.0)
    h0 = lax.dot_general(
        x_ref[...], w_ref[...], (((1,), (1,)), ((), ())),
        preferred_element_type=jnp.float32,
    ) + b_ref[...]
    hs0_ref[...] = h0 * n
    f1_ref[...] = n * n
    f2_ref[...] = n


def _tc_prep(features, W, b2, deg_t):
    return pl.pallas_call(
        _prep_body,
        grid=(N // _BN,),
        in_specs=[
            pl.BlockSpec((_BN, D), lambda i: (i, 0)),
            pl.BlockSpec((D, D), lambda i: (0, 0)),
            pl.BlockSpec((1, D), lambda i: (0, 0)),
            pl.BlockSpec((_BN, NT), lambda i: (i, 0)),
        ],
        out_specs=[
            pl.BlockSpec((_BN, D), lambda i: (i, 0)),
            pl.BlockSpec((_BN, 1), lambda i: (i, 0)),
            pl.BlockSpec((_BN, 1), lambda i: (i, 0)),
        ],
        out_shape=[
            jax.ShapeDtypeStruct((N, D), jnp.float32),
            jax.ShapeDtypeStruct((N, 1), jnp.float32),
            jax.ShapeDtypeStruct((N, 1), jnp.float32),
        ],
    )(features, W, b2, deg_t)


@functools.partial(
    pl.kernel,
    out_type=jax.ShapeDtypeStruct((NT, N4), jnp.float32),
    mesh=_mesh,
    compiler_params=_sc_params,
    scratch_types=[
        pltpu.VMEM((N4,), jnp.float32),
        pltpu.VMEM((N4,), jnp.float32),
        pltpu.VMEM((N,), jnp.float32),
        pltpu.VMEM((N,), jnp.float32),
        pltpu.VMEM((CH,), jnp.int32),
        pltpu.VMEM((CH,), jnp.int32),
        pltpu.VMEM((CH,), jnp.int32),
        pltpu.VMEM((CH,), jnp.int32),
        pltpu.SemaphoreType.DMA,
        pltpu.SemaphoreType.DMA,
    ],
)
def _prop_kernel(hs0_hbm, src_hbm, dst_hbm, f1_hbm, f2_hbm, out_hbm,
                 bufA, bufB, f1v, f2v, sbuf0, sbuf1, dbuf0, dbuf1, sem0, sem1):
    w = lax.axis_index("s") * NC + lax.axis_index("c")
    sems = (sem0, sem1)
    sbufs = (sbuf0, sbuf1)
    dbufs = (dbuf0, dbuf1)
    roff = (w * NCH) // NT

    pltpu.sync_copy(hs0_hbm.at[w], bufA)
    pltpu.sync_copy(hs0_hbm.at[w], bufB)
    pltpu.sync_copy(f1_hbm, f1v)
    pltpu.sync_copy(f2_hbm, f2v)
    iota = lax.iota(jnp.int32, 16)

    def start(b, ci):
        c = ci + roff
        off = jnp.where(c >= NCH, c - NCH, c) * CH
        pltpu.async_copy(src_hbm.at[pl.ds(off, CH)], sbufs[b], sems[b])
        pltpu.async_copy(dst_hbm.at[pl.ds(off, CH)], dbufs[b], sems[b])

    def wait(b):
        pltpu.make_async_copy(src_hbm.at[pl.ds(0, CH)], sbufs[b], sems[b]).wait()
        pltpu.make_async_copy(dst_hbm.at[pl.ds(0, CH)], dbufs[b], sems[b]).wait()

    def edge_pass(table, acc):
        start(0, 0)
        start(1, 1)

        @pl.loop(0, NCH, step=2)
        def _(base):
            for b in range(2):
                ci = base + b
                wait(b)

                @pl.loop(0, CH // 16, unroll=4)
                def _(g):
                    s = sbufs[b][pl.ds(g * 16, 16)]
                    d = dbufs[b][pl.ds(g * 16, 16)]
                    si = s * CPT
                    di = d * CPT
                    for j in range(CPT):
                        v = plsc.load_gather(table, [si + j])
                        plsc.addupdate_scatter(acc, [di + j], v)

                @pl.when(ci + 2 < NCH)
                def _():
                    start(b, ci + 2)

    def scale(acc, fv):
        @functools.partial(plsc.parallel_loop, 0, N4 // 16, unroll=8)
        def _(g):
            base = g * 16
            ridx = lax.shift_right_logical(base + iota, 2)
            f = plsc.load_gather(fv, [ridx])
            acc[pl.ds(base, 16)] = acc[pl.ds(base, 16)] * f

    edge_pass(bufA, bufB)
    scale(bufB, f1v)
    edge_pass(bufB, bufA)
    scale(bufA, f2v)
    pltpu.sync_copy(bufA, out_hbm.at[w])


@jax.jit
def kernel(features, edge_index, W, b):
    src = edge_index[0]
    dst = edge_index[1]
    deg_part = _deg_kernel(dst)
    hs0, f1, f2 = _tc_prep(features, W, b.reshape(1, D), deg_part.T)
    hs0_l = hs0.reshape(N, NT, CPT).transpose(1, 0, 2).reshape(NT, N4)
    out_l = _prop_kernel(hs0_l, src, dst, f1.reshape(N), f2.reshape(N))
    return out_l.reshape(NT, N, CPT).transpose(1, 0, 2).reshape(N, D)


# speedup vs baseline: 4.3398x; 1.9050x over previous
import functools

import jax
import jax.numpy as jnp
from jax import lax
from jax.experimental import pallas as pl
from jax.experimental.pallas import tpu as pltpu
from jax.experimental.pallas import tpu_sc as plsc

N = 10000
D = 128
E = 320000
NC = 2
NS = 16
NT = NC * NS
CPT = D // NT
N4 = N * CPT
EPT = E // NT
CH = 4000
NCH = E // CH

_mesh = plsc.VectorSubcoreMesh(core_axis_name="c", subcore_axis_name="s")
_sc_params = pltpu.CompilerParams(needs_layout_passes=False)


@functools.partial(
    pl.kernel,
    out_type=jax.ShapeDtypeStruct((NT, N), jnp.float32),
    mesh=_mesh,
    compiler_params=_sc_params,
    scratch_types=[
        pltpu.VMEM((N,), jnp.float32),
        pltpu.VMEM((EPT,), jnp.int32),
    ],
)
def _deg_kernel(dst_hbm, out_hbm, cnt_v, dbuf_v):
    w = lax.axis_index("s") * NC + lax.axis_index("c")
    zeros = jnp.zeros((16,), jnp.float32)
    ones = jnp.full((16,), 1.0, jnp.float32)

    @functools.partial(plsc.parallel_loop, 0, N // 16, unroll=8)
    def _(i):
        cnt_v[pl.ds(i * 16, 16)] = zeros

    pltpu.sync_copy(dst_hbm.at[pl.ds(w * EPT, EPT)], dbuf_v)

    @pl.loop(0, EPT // 16, unroll=8)
    def _(g):
        d = dbuf_v[pl.ds(g * 16, 16)]
        plsc.addupdate_scatter(cnt_v, [d], ones)

    pltpu.sync_copy(cnt_v, out_hbm.at[w])


_BN = 1000


def _prep_body(x_ref, w_ref, b_ref, degt_ref, hs0_ref, f1_ref, f2_ref):
    deg = jnp.sum(degt_ref[...], axis=1, keepdims=True)
    n = lax.rsqrt(deg + 1.0)
    h0 = lax.dot_general(
        x_ref[...], w_ref[...], (((1,), (1,)), ((), ())),
        preferred_element_type=jnp.float32,
    ) + b_ref[...]
    hs0_ref[...] = h0 * n
    f1_ref[...] = n * n
    f2_ref[...] = n


def _tc_prep(features, W, b2, deg_t):
    return pl.pallas_call(
        _prep_body,
        grid=(N // _BN,),
        in_specs=[
            pl.BlockSpec((_BN, D), lambda i: (i, 0)),
            pl.BlockSpec((D, D), lambda i: (0, 0)),
            pl.BlockSpec((1, D), lambda i: (0, 0)),
            pl.BlockSpec((_BN, NT), lambda i: (i, 0)),
        ],
        out_specs=[
            pl.BlockSpec((_BN, D), lambda i: (i, 0)),
            pl.BlockSpec((_BN, 1), lambda i: (i, 0)),
            pl.BlockSpec((_BN, 1), lambda i: (i, 0)),
        ],
        out_shape=[
            jax.ShapeDtypeStruct((N, D), jnp.float32),
            jax.ShapeDtypeStruct((N, 1), jnp.float32),
            jax.ShapeDtypeStruct((N, 1), jnp.float32),
        ],
    )(features, W, b2, deg_t)


@functools.partial(
    pl.kernel,
    out_type=jax.ShapeDtypeStruct((NT, N4), jnp.float32),
    mesh=_mesh,
    compiler_params=_sc_params,
    scratch_types=[
        pltpu.VMEM((N4,), jnp.float32),
        pltpu.VMEM((N4,), jnp.float32),
        pltpu.VMEM((N,), jnp.float32),
        pltpu.VMEM((N,), jnp.float32),
        pltpu.VMEM((CH,), jnp.int32),
        pltpu.VMEM((CH,), jnp.int32),
        pltpu.VMEM((CH,), jnp.int32),
        pltpu.VMEM((CH,), jnp.int32),
        pltpu.SemaphoreType.DMA,
        pltpu.SemaphoreType.DMA,
    ],
)
def _prop_kernel(hs0_hbm, src_hbm, dst_hbm, f1_hbm, f2_hbm, out_hbm,
                 bufA, bufB, f1v, f2v, sbuf0, sbuf1, dbuf0, dbuf1, sem0, sem1):
    w = lax.axis_index("s") * NC + lax.axis_index("c")
    sems = (sem0, sem1)
    sbufs = (sbuf0, sbuf1)
    dbufs = (dbuf0, dbuf1)
    roff = (w * NCH) // NT

    pltpu.sync_copy(hs0_hbm.at[w], bufA)
    pltpu.sync_copy(hs0_hbm.at[w], bufB)
    pltpu.sync_copy(f1_hbm, f1v)
    pltpu.sync_copy(f2_hbm, f2v)
    iota = lax.iota(jnp.int32, 16)

    def start(b, ci):
        c = ci + roff
        off = jnp.where(c >= NCH, c - NCH, c) * CH
        pltpu.async_copy(src_hbm.at[pl.ds(off, CH)], sbufs[b], sems[b])
        pltpu.async_copy(dst_hbm.at[pl.ds(off, CH)], dbufs[b], sems[b])

    def wait(b):
        pltpu.make_async_copy(src_hbm.at[pl.ds(0, CH)], sbufs[b], sems[b]).wait()
        pltpu.make_async_copy(dst_hbm.at[pl.ds(0, CH)], dbufs[b], sems[b]).wait()

    def edge_pass(table, acc):
        start(0, 0)
        start(1, 1)

        @pl.loop(0, NCH, step=2)
        def _(base):
            for b in range(2):
                ci = base + b
                wait(b)

                @pl.loop(0, CH // 32, unroll=2)
                def _(g):
                    base = g * 32
                    ss = [sbufs[b][pl.ds(base + 16 * k, 16)] for k in (0, 1)]
                    ds_ = [dbufs[b][pl.ds(base + 16 * k, 16)] for k in (0, 1)]
                    sis = [s * CPT for s in ss]
                    dis = [d * CPT for d in ds_]
                    vals = [plsc.load_gather(table, [sis[k] + j])
                            for k in (0, 1) for j in range(CPT)]
                    for k in (0, 1):
                        for j in range(CPT):
                            plsc.addupdate_scatter(
                                acc, [dis[k] + j], vals[k * CPT + j])

                @pl.when(ci + 2 < NCH)
                def _():
                    start(b, ci + 2)

    def scale(acc, fv):
        @functools.partial(plsc.parallel_loop, 0, N4 // 16, unroll=8)
        def _(g):
            base = g * 16
            ridx = lax.shift_right_logical(base + iota, 2)
            f = plsc.load_gather(fv, [ridx])
            acc[pl.ds(base, 16)] = acc[pl.ds(base, 16)] * f

    edge_pass(bufA, bufB)
    scale(bufB, f1v)
    edge_pass(bufB, bufA)
    scale(bufA, f2v)
    pltpu.sync_copy(bufA, out_hbm.at[w])


@jax.jit
def kernel(features, edge_index, W, b):
    src = edge_index[0]
    dst = edge_index[1]
    deg_part = _deg_kernel(dst)
    hs0, f1, f2 = _tc_prep(features, W, b.reshape(1, D), deg_part.T)
    hs0_l = hs0.reshape(N, NT, CPT).transpose(1, 0, 2).reshape(NT, N4)
    out_l = _prop_kernel(hs0_l, src, dst, f1.reshape(N), f2.reshape(N))
    return out_l.reshape(NT, N, CPT).transpose(1, 0, 2).reshape(N, D)


# speedup vs baseline: 4.6428x; 1.0698x over previous
import functools

import jax
import jax.numpy as jnp
from jax import lax
from jax.experimental import pallas as pl
from jax.experimental.pallas import tpu as pltpu
from jax.experimental.pallas import tpu_sc as plsc

N = 10000
D = 128
E = 320000
NC = 2
NS = 16
NT = NC * NS
CPT = D // NT
N4 = N * CPT
EPT = E // NT
CH = 3200
NCH = E // CH

_mesh = plsc.VectorSubcoreMesh(core_axis_name="c", subcore_axis_name="s")
_sc_params = pltpu.CompilerParams(needs_layout_passes=False)


@functools.partial(
    pl.kernel,
    out_type=jax.ShapeDtypeStruct((NT, N), jnp.float32),
    mesh=_mesh,
    compiler_params=_sc_params,
    scratch_types=[
        pltpu.VMEM((N,), jnp.float32),
        pltpu.VMEM((EPT,), jnp.int32),
    ],
)
def _deg_kernel(dst_hbm, out_hbm, cnt_v, dbuf_v):
    w = lax.axis_index("s") * NC + lax.axis_index("c")
    zeros = jnp.zeros((16,), jnp.float32)
    ones = jnp.full((16,), 1.0, jnp.float32)

    @functools.partial(plsc.parallel_loop, 0, N // 16, unroll=8)
    def _(i):
        cnt_v[pl.ds(i * 16, 16)] = zeros

    pltpu.sync_copy(dst_hbm.at[pl.ds(w * EPT, EPT)], dbuf_v)

    @pl.loop(0, EPT // 16, unroll=8)
    def _(g):
        d = dbuf_v[pl.ds(g * 16, 16)]
        plsc.addupdate_scatter(cnt_v, [d], ones)

    pltpu.sync_copy(cnt_v, out_hbm.at[w])


_BN = 1000


def _prep_body(x_ref, w_ref, b_ref, degt_ref, hs0_ref, f1_ref, f2_ref):
    deg = jnp.sum(degt_ref[...], axis=1, keepdims=True)
    n = lax.rsqrt(deg + 1.0)
    h0 = lax.dot_general(
        x_ref[...], w_ref[...], (((1,), (1,)), ((), ())),
        preferred_element_type=jnp.float32,
    ) + b_ref[...]
    hs0_ref[...] = h0 * n
    f1_ref[...] = n * n
    f2_ref[...] = n


def _tc_prep(features, W, b2, deg_t):
    return pl.pallas_call(
        _prep_body,
        grid=(N // _BN,),
        in_specs=[
            pl.BlockSpec((_BN, D), lambda i: (i, 0)),
            pl.BlockSpec((D, D), lambda i: (0, 0)),
            pl.BlockSpec((1, D), lambda i: (0, 0)),
            pl.BlockSpec((_BN, NT), lambda i: (i, 0)),
        ],
        out_specs=[
            pl.BlockSpec((_BN, D), lambda i: (i, 0)),
            pl.BlockSpec((_BN, 1), lambda i: (i, 0)),
            pl.BlockSpec((_BN, 1), lambda i: (i, 0)),
        ],
        out_shape=[
            jax.ShapeDtypeStruct((N, D), jnp.float32),
            jax.ShapeDtypeStruct((N, 1), jnp.float32),
            jax.ShapeDtypeStruct((N, 1), jnp.float32),
        ],
    )(features, W, b2, deg_t)


@functools.partial(
    pl.kernel,
    out_type=jax.ShapeDtypeStruct((NT, N4), jnp.float32),
    mesh=_mesh,
    compiler_params=_sc_params,
    scratch_types=[
        pltpu.VMEM((N4,), jnp.float32),
        pltpu.VMEM((N4,), jnp.float32),
        pltpu.VMEM((N,), jnp.float32),
        pltpu.VMEM((N,), jnp.float32),
        pltpu.VMEM((CH,), jnp.int32),
        pltpu.VMEM((CH,), jnp.int32),
        pltpu.VMEM((CH,), jnp.int32),
        pltpu.VMEM((CH,), jnp.int32),
        pltpu.SemaphoreType.DMA,
        pltpu.SemaphoreType.DMA,
    ],
)
def _prop_kernel(hs0_hbm, src_hbm, dst_hbm, f1_hbm, f2_hbm, out_hbm,
                 bufA, bufB, f1v, f2v, sbuf0, sbuf1, dbuf0, dbuf1, sem0, sem1):
    w = lax.axis_index("s") * NC + lax.axis_index("c")
    sems = (sem0, sem1)
    sbufs = (sbuf0, sbuf1)
    dbufs = (dbuf0, dbuf1)
    roff = (w * NCH) // NT

    pltpu.sync_copy(hs0_hbm.at[w], bufA)
    pltpu.sync_copy(hs0_hbm.at[w], bufB)
    pltpu.sync_copy(f1_hbm, f1v)
    pltpu.sync_copy(f2_hbm, f2v)
    iota = lax.iota(jnp.int32, 16)

    def start(b, ci):
        c = ci + roff
        off = jnp.where(c >= NCH, c - NCH, c) * CH
        pltpu.async_copy(src_hbm.at[pl.ds(off, CH)], sbufs[b], sems[b])
        pltpu.async_copy(dst_hbm.at[pl.ds(off, CH)], dbufs[b], sems[b])

    def wait(b):
        pltpu.make_async_copy(src_hbm.at[pl.ds(0, CH)], sbufs[b], sems[b]).wait()
        pltpu.make_async_copy(dst_hbm.at[pl.ds(0, CH)], dbufs[b], sems[b]).wait()

    def edge_pass(table, acc):
        start(0, 0)
        start(1, 1)

        @pl.loop(0, NCH, step=2)
        def _(base):
            for b in range(2):
                ci = base + b
                wait(b)

                @pl.loop(0, CH // 64, unroll=1)
                def _(g):
                    base = g * 64
                    ks = range(4)
                    ss = [sbufs[b][pl.ds(base + 16 * k, 16)] for k in ks]
                    ds_ = [dbufs[b][pl.ds(base + 16 * k, 16)] for k in ks]
                    sis = [s * CPT for s in ss]
                    dis = [d * CPT for d in ds_]
                    vals = [plsc.load_gather(table, [sis[k] + j])
                            for k in ks for j in range(CPT)]
                    for k in ks:
                        for j in range(CPT):
                            plsc.addupdate_scatter(
                                acc, [dis[k] + j], vals[k * CPT + j])

                @pl.when(ci + 2 < NCH)
                def _():
                    start(b, ci + 2)

    def scale(acc, fv):
        @functools.partial(plsc.parallel_loop, 0, N4 // 16, unroll=8)
        def _(g):
            base = g * 16
            ridx = lax.shift_right_logical(base + iota, 2)
            f = plsc.load_gather(fv, [ridx])
            acc[pl.ds(base, 16)] = acc[pl.ds(base, 16)] * f

    edge_pass(bufA, bufB)
    scale(bufB, f1v)
    edge_pass(bufB, bufA)
    scale(bufA, f2v)
    pltpu.sync_copy(bufA, out_hbm.at[w])


@jax.jit
def kernel(features, edge_index, W, b):
    src = edge_index[0]
    dst = edge_index[1]
    deg_part = _deg_kernel(dst)
    hs0, f1, f2 = _tc_prep(features, W, b.reshape(1, D), deg_part.T)
    hs0_l = hs0.reshape(N, NT, CPT).transpose(1, 0, 2).reshape(NT, N4)
    out_l = _prop_kernel(hs0_l, src, dst, f1.reshape(N), f2.reshape(N))
    return out_l.reshape(NT, N, CPT).transpose(1, 0, 2).reshape(N, D)


# speedup vs baseline: 4.7598x; 1.0252x over previous
import functools

import jax
import jax.numpy as jnp
from jax import lax
from jax.experimental import pallas as pl
from jax.experimental.pallas import tpu as pltpu
from jax.experimental.pallas import tpu_sc as plsc

N = 10000
D = 128
E = 320000
NC = 2
NS = 16
NT = NC * NS
CPT = D // NT
N4 = N * CPT
EPT = E // NT
CH = 4000
NCH = E // CH

_mesh = plsc.VectorSubcoreMesh(core_axis_name="c", subcore_axis_name="s")
_sc_params = pltpu.CompilerParams(needs_layout_passes=False)


@functools.partial(
    pl.kernel,
    out_type=jax.ShapeDtypeStruct((NT, N), jnp.float32),
    mesh=_mesh,
    compiler_params=_sc_params,
    scratch_types=[
        pltpu.VMEM((N,), jnp.float32),
        pltpu.VMEM((EPT,), jnp.int32),
    ],
)
def _deg_kernel(dst_hbm, out_hbm, cnt_v, dbuf_v):
    w = lax.axis_index("s") * NC + lax.axis_index("c")
    zeros = jnp.zeros((16,), jnp.float32)
    ones = jnp.full((16,), 1.0, jnp.float32)

    @functools.partial(plsc.parallel_loop, 0, N // 16, unroll=8)
    def _(i):
        cnt_v[pl.ds(i * 16, 16)] = zeros

    pltpu.sync_copy(dst_hbm.at[pl.ds(w * EPT, EPT)], dbuf_v)

    @pl.loop(0, EPT // 16, unroll=8)
    def _(g):
        d = dbuf_v[pl.ds(g * 16, 16)]
        plsc.addupdate_scatter(cnt_v, [d], ones)

    pltpu.sync_copy(cnt_v, out_hbm.at[w])


_BN = 1000


def _prep_body(x_ref, w_ref, b_ref, degt_ref, hs0_ref, f1_ref, f2_ref):
    deg = jnp.sum(degt_ref[...], axis=1, keepdims=True)
    n = lax.rsqrt(deg + 1.0)
    h0 = lax.dot_general(
        x_ref[...], w_ref[...], (((1,), (1,)), ((), ())),
        preferred_element_type=jnp.float32,
    ) + b_ref[...]
    hs0_ref[...] = h0 * n
    f1_ref[...] = n * n
    f2_ref[...] = n


def _tc_prep(features, W, b2, deg_t):
    return pl.pallas_call(
        _prep_body,
        grid=(N // _BN,),
        in_specs=[
            pl.BlockSpec((_BN, D), lambda i: (i, 0)),
            pl.BlockSpec((D, D), lambda i: (0, 0)),
            pl.BlockSpec((1, D), lambda i: (0, 0)),
            pl.BlockSpec((_BN, NT), lambda i: (i, 0)),
        ],
        out_specs=[
            pl.BlockSpec((_BN, D), lambda i: (i, 0)),
            pl.BlockSpec((_BN, 1), lambda i: (i, 0)),
            pl.BlockSpec((_BN, 1), lambda i: (i, 0)),
        ],
        out_shape=[
            jax.ShapeDtypeStruct((N, D), jnp.float32),
            jax.ShapeDtypeStruct((N, 1), jnp.float32),
            jax.ShapeDtypeStruct((N, 1), jnp.float32),
        ],
    )(features, W, b2, deg_t)


@functools.partial(
    pl.kernel,
    out_type=jax.ShapeDtypeStruct((NT, N4), jnp.float32),
    mesh=_mesh,
    compiler_params=_sc_params,
    scratch_types=[
        pltpu.VMEM((N4,), jnp.float32),
        pltpu.VMEM((N4,), jnp.float32),
        pltpu.VMEM((N,), jnp.float32),
        pltpu.VMEM((N,), jnp.float32),
        pltpu.VMEM((CH + 32,), jnp.int32),
        pltpu.VMEM((CH + 32,), jnp.int32),
        pltpu.VMEM((CH + 32,), jnp.int32),
        pltpu.VMEM((CH + 32,), jnp.int32),
        pltpu.SemaphoreType.DMA,
        pltpu.SemaphoreType.DMA,
    ],
)
def _prop_kernel(hs0_hbm, src_hbm, dst_hbm, f1_hbm, f2_hbm, out_hbm,
                 bufA, bufB, f1v, f2v, sbuf0, sbuf1, dbuf0, dbuf1, sem0, sem1):
    w = lax.axis_index("s") * NC + lax.axis_index("c")
    sems = (sem0, sem1)
    sbufs = (sbuf0, sbuf1)
    dbufs = (dbuf0, dbuf1)
    roff = (w * NCH) // NT

    pltpu.sync_copy(hs0_hbm.at[w], bufA)
    pltpu.sync_copy(hs0_hbm.at[w], bufB)
    pltpu.sync_copy(f1_hbm, f1v)
    pltpu.sync_copy(f2_hbm, f2v)
    iota = lax.iota(jnp.int32, 16)

    def start(b, ci):
        c = ci + roff
        off = jnp.where(c >= NCH, c - NCH, c) * CH
        pltpu.async_copy(src_hbm.at[pl.ds(off, CH)], sbufs[b].at[pl.ds(0, CH)],
                         sems[b])
        pltpu.async_copy(dst_hbm.at[pl.ds(off, CH)], dbufs[b].at[pl.ds(0, CH)],
                         sems[b])

    def wait(b):
        pltpu.make_async_copy(src_hbm.at[pl.ds(0, CH)],
                              sbufs[b].at[pl.ds(0, CH)], sems[b]).wait()
        pltpu.make_async_copy(dst_hbm.at[pl.ds(0, CH)],
                              dbufs[b].at[pl.ds(0, CH)], sems[b]).wait()

    def edge_pass(table, acc):
        start(0, 0)
        start(1, 1)

        @pl.loop(0, NCH, step=2)
        def _(base):
            for b in range(2):
                ci = base + b
                wait(b)

                def idx_at(e):
                    return ([sbufs[b][pl.ds(e + 16 * k, 16)] for k in (0, 1)]
                            + [dbufs[b][pl.ds(e + 16 * k, 16)]
                               for k in (0, 1)])

                @pl.loop(0, CH // 32, init_carry=tuple(idx_at(0)), unroll=2)
                def _(g, carry):
                    s0, s1, d0, d1 = carry
                    sis = [s0 * CPT, s1 * CPT]
                    dis = [d0 * CPT, d1 * CPT]
                    vals = [plsc.load_gather(table, [sis[k] + j])
                            for k in (0, 1) for j in range(CPT)]
                    nxt = tuple(idx_at(g * 32 + 32))
                    for k in (0, 1):
                        for j in range(CPT):
                            plsc.addupdate_scatter(
                                acc, [dis[k] + j], vals[k * CPT + j])
                    return nxt

                @pl.when(ci + 2 < NCH)
                def _():
                    start(b, ci + 2)

    def scale(acc, fv):
        @functools.partial(plsc.parallel_loop, 0, N4 // 16, unroll=8)
        def _(g):
            base = g * 16
            ridx = lax.shift_right_logical(base + iota, 2)
            f = plsc.load_gather(fv, [ridx])
            acc[pl.ds(base, 16)] = acc[pl.ds(base, 16)] * f

    edge_pass(bufA, bufB)
    scale(bufB, f1v)
    edge_pass(bufB, bufA)
    scale(bufA, f2v)
    pltpu.sync_copy(bufA, out_hbm.at[w])


@jax.jit
def kernel(features, edge_index, W, b):
    src = edge_index[0]
    dst = edge_index[1]
    deg_part = _deg_kernel(dst)
    hs0, f1, f2 = _tc_prep(features, W, b.reshape(1, D), deg_part.T)
    hs0_l = hs0.reshape(N, NT, CPT).transpose(1, 0, 2).reshape(NT, N4)
    out_l = _prop_kernel(hs0_l, src, dst, f1.reshape(N), f2.reshape(N))
    return out_l.reshape(NT, N, CPT).transpose(1, 0, 2).reshape(N, D)
